# Initial kernel scaffold; baseline (speedup 1.0000x reference)
import jax
import jax.numpy as jnp
from jax.experimental import pallas as pl


def kernel(x, paths, W_ih, W_hh, b_ih, b_hh, gamma, beta):
    raise NotImplementedError("write your pallas kernel here")


# speedup vs baseline: 1.5616x; 1.5616x over previous
import functools

import jax
import jax.numpy as jnp
from jax import lax
from jax.experimental import pallas as pl
from jax.experimental.pallas import tpu as pltpu
from jax.experimental.pallas import tpu_sc as plsc

_NC = 2
_NS = 16


def _sc_gather(x, idx3):
    nw, n_chunks, chunk = idx3.shape
    d = x.shape[1]
    total = nw * n_chunks * chunk
    per_w = n_chunks * chunk
    mesh = plsc.VectorSubcoreMesh(core_axis_name="c", subcore_axis_name="s")

    @functools.partial(
        pl.kernel,
        out_type=jax.ShapeDtypeStruct((total, d), x.dtype),
        mesh=mesh,
        scratch_types=[
            pltpu.VMEM((n_chunks, chunk), jnp.int32),
            pltpu.VMEM((chunk, d), x.dtype),
            pltpu.SemaphoreType.DMA,
        ],
    )
    def k(x_hbm, idx_hbm, out_hbm, idx_v, buf_v, sem):
        wid = lax.axis_index("s") * _NC + lax.axis_index("c")
        base = wid * per_w
        pltpu.sync_copy(idx_hbm.at[wid], idx_v)

        @pl.loop(0, n_chunks)
        def _(j):
            pltpu.async_copy(x_hbm.at[idx_v.at[j]], buf_v, sem).wait()
            pltpu.sync_copy(buf_v, out_hbm.at[pl.ds(base + j * chunk, chunk)])

    return k(x, idx3)


def _sc_scatter_residual(h_t, dst3, x):
    n, d = x.shape
    dh = d // _NC
    ns, n_chunks, chunk = dst3.shape
    per_s = n_chunks * chunk
    rows_main = (n // ns) // 8 * 8
    tail_base = ns * rows_main
    tail_rows = n - tail_base
    mesh = plsc.VectorSubcoreMesh(core_axis_name="c", subcore_axis_name="s")

    @functools.partial(
        pl.kernel,
        out_type=jax.ShapeDtypeStruct((n, d), x.dtype),
        mesh=mesh,
        scratch_types=[
            pltpu.VMEM((n_chunks, chunk), jnp.int32),
            pltpu.VMEM((chunk, dh), x.dtype),
            pltpu.VMEM_SHARED((n, dh), x.dtype),
        ],
    )
    def k(h_hbm, dst_hbm, x_hbm, out_hbm, idx_v, buf_v, acc_sh):
        c = lax.axis_index("c")
        s = lax.axis_index("s")
        col0 = c * dh
        r0 = s * rows_main
        pltpu.sync_copy(
            x_hbm.at[pl.ds(r0, rows_main), pl.ds(col0, dh)],
            acc_sh.at[pl.ds(r0, rows_main)],
        )
        if tail_rows:
            @pl.when(s == ns - 1)
            def _():
                pltpu.sync_copy(
                    x_hbm.at[pl.ds(tail_base, tail_rows), pl.ds(col0, dh)],
                    acc_sh.at[pl.ds(tail_base, tail_rows)],
                )
        pltpu.sync_copy(dst_hbm.at[s], idx_v)
        plsc.subcore_barrier()

        @pl.loop(0, n_chunks)
        def _(j):
            rbase = s * per_s + j * chunk
            pltpu.sync_copy(h_hbm.at[pl.ds(rbase, chunk), pl.ds(col0, dh)], buf_v)
            pltpu.sync_copy(buf_v, acc_sh.at[idx_v.at[j]], add=True)

        plsc.subcore_barrier()
        pltpu.sync_copy(
            acc_sh.at[pl.ds(r0, rows_main)],
            out_hbm.at[pl.ds(r0, rows_main), pl.ds(col0, dh)],
        )
        if tail_rows:
            @pl.when(s == ns - 1)
            def _():
                pltpu.sync_copy(
                    acc_sh.at[pl.ds(tail_base, tail_rows)],
                    out_hbm.at[pl.ds(tail_base, tail_rows), pl.ds(col0, dh)],
                )

    return k(h_t, dst3, x)


def _tc_lstm(seq, w_ih, w_hh, bias, blk):
    p, ld = seq.shape
    g, d = w_ih.shape
    steps = ld // d
    prec = lax.Precision.HIGHEST
    dn = (((1,), (1,)), ((), ()))

    def body(seq_ref, wih_ref, whh_ref, b_ref, out_ref):
        wih = wih_ref[...]
        whh = whh_ref[...]
        b = b_ref[...]
        s = seq_ref[...]
        h = None
        c = None
        for t in range(steps):
            st = s[:, t * d:(t + 1) * d]
            gates = lax.dot_general(st, wih, dn, precision=prec,
                                    preferred_element_type=jnp.float32) + b
            if h is not None:
                gates = gates + lax.dot_general(h, whh, dn, precision=prec,
                                                preferred_element_type=jnp.float32)
            gi = jax.nn.sigmoid(gates[:, 0 * d:1 * d])
            gf = jax.nn.sigmoid(gates[:, 1 * d:2 * d])
            gg = jnp.tanh(gates[:, 2 * d:3 * d])
            go = jax.nn.sigmoid(gates[:, 3 * d:4 * d])
            c = gi * gg if c is None else gf * c + gi * gg
            h = go * jnp.tanh(c)
        out_ref[...] = h

    return pl.pallas_call(
        body,
        grid=(p // blk,),
        in_specs=[
            pl.BlockSpec((blk, ld), lambda i: (i, 0)),
            pl.BlockSpec((g, d), lambda i: (0, 0)),
            pl.BlockSpec((g, d), lambda i: (0, 0)),
            pl.BlockSpec((1, g), lambda i: (0, 0)),
        ],
        out_specs=pl.BlockSpec((blk, d), lambda i: (i, 0)),
        out_shape=jax.ShapeDtypeStruct((p, d), seq.dtype),
    )(seq, w_ih, w_hh, bias)


def _tc_bn_relu(y, gamma, beta):
    n, d = y.shape

    def body(y_ref, g_ref, b_ref, o_ref):
        v = y_ref[...]
        mean = jnp.mean(v, axis=0, keepdims=True)
        cent = v - mean
        var = jnp.mean(cent * cent, axis=0, keepdims=True)
        scaled = cent * lax.rsqrt(var + 1e-5) * g_ref[...] + b_ref[...]
        o_ref[...] = jnp.maximum(scaled, 0.0)

    return pl.pallas_call(
        body,
        out_shape=jax.ShapeDtypeStruct((n, d), y.dtype),
    )(y, gamma.reshape(1, d), beta.reshape(1, d))


def kernel(x, paths, W_ih, W_hh, b_ih, b_hh, gamma, beta):
    n, d = x.shape
    p, l = paths.shape
    paths = paths.astype(jnp.int32)
    bias = (b_ih + b_hh).reshape(1, 4 * d).astype(jnp.float32)

    nw = _NC * _NS
    chunk = 80
    per_w = (p * l) // nw
    idx3 = paths.reshape(nw, per_w // chunk, chunk)
    seq = _sc_gather(x, idx3)
    seq = seq.reshape(p, l * d)

    h_t = _tc_lstm(seq, W_ih, W_hh, bias, blk=1000)

    chunk2 = 80
    per_s = p // _NS
    dst3 = paths[:, l - 1].reshape(_NS, per_s // chunk2, chunk2)
    y = _sc_scatter_residual(h_t, dst3, x)

    return _tc_bn_relu(y, gamma, beta)


# speedup vs baseline: 3.3204x; 2.1263x over previous
import functools

import jax
import jax.numpy as jnp
from jax import lax
from jax.experimental import pallas as pl
from jax.experimental.pallas import tpu as pltpu
from jax.experimental.pallas import tpu_sc as plsc

_NC = 2
_NS = 16


def _sc_gather(x, idx3):
    nw, n_chunks, chunk = idx3.shape
    d = x.shape[1]
    total = nw * n_chunks * chunk
    per_w = n_chunks * chunk
    mesh = plsc.VectorSubcoreMesh(core_axis_name="c", subcore_axis_name="s")

    @functools.partial(
        pl.kernel,
        out_type=jax.ShapeDtypeStruct((total, d), x.dtype),
        mesh=mesh,
        scratch_types=[
            pltpu.VMEM((n_chunks, chunk), jnp.int32),
            pltpu.VMEM((chunk, d), x.dtype),
            pltpu.SemaphoreType.DMA,
        ],
    )
    def k(x_hbm, idx_hbm, out_hbm, idx_v, buf_v, sem):
        wid = lax.axis_index("s") * _NC + lax.axis_index("c")
        base = wid * per_w
        pltpu.sync_copy(idx_hbm.at[wid], idx_v)

        @pl.loop(0, n_chunks)
        def _(j):
            pltpu.async_copy(x_hbm.at[idx_v.at[j]], buf_v, sem).wait()
            pltpu.sync_copy(buf_v, out_hbm.at[pl.ds(base + j * chunk, chunk)])

    return k(x, idx3)


def _sc_scatter_residual(h_t, dst3, x):
    n, d = x.shape
    dh = d // _NC
    ns, n_chunks, chunk = dst3.shape
    per_s = n_chunks * chunk
    rows_main = (n // ns) // 8 * 8
    tail_base = ns * rows_main
    tail_rows = n - tail_base
    mesh = plsc.VectorSubcoreMesh(core_axis_name="c", subcore_axis_name="s")

    @functools.partial(
        pl.kernel,
        out_type=jax.ShapeDtypeStruct((n, d), x.dtype),
        mesh=mesh,
        scratch_types=[
            pltpu.VMEM((n_chunks, chunk), jnp.int32),
            pltpu.VMEM((chunk, dh), x.dtype),
            pltpu.VMEM_SHARED((n, dh), x.dtype),
        ],
    )
    def k(h_hbm, dst_hbm, x_hbm, out_hbm, idx_v, buf_v, acc_sh):
        c = lax.axis_index("c")
        s = lax.axis_index("s")
        col0 = c * dh
        r0 = s * rows_main
        pltpu.sync_copy(
            x_hbm.at[pl.ds(r0, rows_main), pl.ds(col0, dh)],
            acc_sh.at[pl.ds(r0, rows_main)],
        )
        if tail_rows:
            @pl.when(s == ns - 1)
            def _():
                pltpu.sync_copy(
                    x_hbm.at[pl.ds(tail_base, tail_rows), pl.ds(col0, dh)],
                    acc_sh.at[pl.ds(tail_base, tail_rows)],
                )
        pltpu.sync_copy(dst_hbm.at[s], idx_v)
        plsc.subcore_barrier()

        @pl.loop(0, n_chunks)
        def _(j):
            rbase = s * per_s + j * chunk
            pltpu.sync_copy(h_hbm.at[pl.ds(rbase, chunk), pl.ds(col0, dh)], buf_v)
            pltpu.sync_copy(buf_v, acc_sh.at[idx_v.at[j]], add=True)

        plsc.subcore_barrier()
        pltpu.sync_copy(
            acc_sh.at[pl.ds(r0, rows_main)],
            out_hbm.at[pl.ds(r0, rows_main), pl.ds(col0, dh)],
        )
        if tail_rows:
            @pl.when(s == ns - 1)
            def _():
                pltpu.sync_copy(
                    acc_sh.at[pl.ds(tail_base, tail_rows)],
                    out_hbm.at[pl.ds(tail_base, tail_rows), pl.ds(col0, dh)],
                )

    return k(h_t, dst3, x)


def _tc_lstm(seq, w_ih, w_hh, bias, blk):
    p, ld = seq.shape
    g, d = w_ih.shape
    steps = ld // d
    prec = lax.Precision.DEFAULT
    dn = (((1,), (1,)), ((), ()))

    def body(seq_ref, wih_ref, whh_ref, b_ref, out_ref):
        wih = wih_ref[...]
        whh = whh_ref[...]
        b = b_ref[...]
        s = seq_ref[...]
        h = None
        c = None
        for t in range(steps):
            st = s[:, t * d:(t + 1) * d]
            gates = lax.dot_general(st, wih, dn, precision=prec,
                                    preferred_element_type=jnp.float32) + b
            if h is not None:
                gates = gates + lax.dot_general(h, whh, dn, precision=prec,
                                                preferred_element_type=jnp.float32)
            gi = jax.nn.sigmoid(gates[:, 0 * d:1 * d])
            gf = jax.nn.sigmoid(gates[:, 1 * d:2 * d])
            gg = jnp.tanh(gates[:, 2 * d:3 * d])
            go = jax.nn.sigmoid(gates[:, 3 * d:4 * d])
            c = gi * gg if c is None else gf * c + gi * gg
            h = go * jnp.tanh(c)
        out_ref[...] = h

    return pl.pallas_call(
        body,
        grid=(p // blk,),
        in_specs=[
            pl.BlockSpec((blk, ld), lambda i: (i, 0)),
            pl.BlockSpec((g, d), lambda i: (0, 0)),
            pl.BlockSpec((g, d), lambda i: (0, 0)),
            pl.BlockSpec((1, g), lambda i: (0, 0)),
        ],
        out_specs=pl.BlockSpec((blk, d), lambda i: (i, 0)),
        out_shape=jax.ShapeDtypeStruct((p, d), seq.dtype),
    )(seq, w_ih, w_hh, bias)


def _tc_bn_relu(y, gamma, beta):
    n, d = y.shape

    def body(y_ref, g_ref, b_ref, o_ref):
        v = y_ref[...]
        mean = jnp.mean(v, axis=0, keepdims=True)
        cent = v - mean
        var = jnp.mean(cent * cent, axis=0, keepdims=True)
        scaled = cent * lax.rsqrt(var + 1e-5) * g_ref[...] + b_ref[...]
        o_ref[...] = jnp.maximum(scaled, 0.0)

    return pl.pallas_call(
        body,
        out_shape=jax.ShapeDtypeStruct((n, d), y.dtype),
    )(y, gamma.reshape(1, d), beta.reshape(1, d))


def kernel(x, paths, W_ih, W_hh, b_ih, b_hh, gamma, beta):
    n, d = x.shape
    p, l = paths.shape
    paths = paths.astype(jnp.int32)
    bias = (b_ih + b_hh).reshape(1, 4 * d).astype(jnp.float32)

    nw = _NC * _NS
    chunk = 80
    per_w = (p * l) // nw
    idx3 = paths.reshape(nw, per_w // chunk, chunk)
    seq = _sc_gather(x, idx3)
    seq = seq.reshape(p, l * d)

    h_t = _tc_lstm(seq, W_ih, W_hh, bias, blk=1000)

    chunk2 = 80
    per_s = p // _NS
    dst3 = paths[:, l - 1].reshape(_NS, per_s // chunk2, chunk2)
    y = _sc_scatter_residual(h_t, dst3, x)

    return _tc_bn_relu(y, gamma, beta)
